# baseline (speedup 1.0000x reference)
import jax
import jax.numpy as jnp
from jax import lax
from jax.experimental import pallas as pl
from jax.experimental.pallas import tpu as pltpu
from jax.experimental.pallas import tpu_sc as plsc

BATCH = 4096
SEQ = 200
EMB_DIM = 128
NUM_WORKERS = 32
ROWS_PER_WORKER = BATCH // NUM_WORKERS
HALF_SEQ = SEQ // 2
NBUF = 4


def _bag_body(text_hbm, emb_hbm, out_hbm, idx_v, rows_v, out_stage,
              sem0, sem1, sem2, sem3):
    wid = lax.axis_index("s") * 2 + lax.axis_index("c")
    base = wid * ROWS_PER_WORKER
    sems = (sem0, sem1, sem2, sem3)

    pltpu.sync_copy(text_hbm.at[pl.ds(base, ROWS_PER_WORKER)], idx_v)

    def issue(r, b):
        pltpu.async_copy(emb_hbm.at[idx_v.at[r, 0]], rows_v.at[b, 0], sems[b])
        pltpu.async_copy(emb_hbm.at[idx_v.at[r, 1]], rows_v.at[b, 1], sems[b])

    def wait(r, b):
        pltpu.make_async_copy(emb_hbm.at[idx_v.at[r, 0]], rows_v.at[b, 0], sems[b]).wait()
        pltpu.make_async_copy(emb_hbm.at[idx_v.at[r, 1]], rows_v.at[b, 1], sems[b]).wait()

    def accum(r, b):
        mask_hi = jnp.int32(-65536)

        def tok_step(t, acc):
            new = list(acc)
            for h in range(2):
                for c in range(4):
                    w = rows_v[b, h, t, pl.ds(c * 16, 16)]
                    hi = lax.bitcast_convert_type(w & mask_hi, jnp.float32)
                    lo = lax.bitcast_convert_type(w << 16, jnp.float32)
                    new[2 * c] = new[2 * c] + lo
                    new[2 * c + 1] = new[2 * c + 1] + hi
            return tuple(new)

        acc = tuple(jnp.zeros((16,), jnp.float32) for _ in range(8))
        acc = lax.fori_loop(0, HALF_SEQ, tok_step, acc)
        rnd = jnp.int32(32768)
        for c in range(4):
            lo_bits = lax.bitcast_convert_type(acc[2 * c], jnp.int32) + rnd
            hi_bits = lax.bitcast_convert_type(acc[2 * c + 1], jnp.int32) + rnd
            word = lax.shift_right_logical(lo_bits, 16) | (hi_bits & mask_hi)
            out_stage[r, pl.ds(c * 16, 16)] = word

    for b in range(NBUF):
        issue(b, b)

    def body(g, _):
        for b in range(NBUF):
            r = NBUF * g + b
            wait(r, b)
            accum(r, b)
            issue(r + NBUF, b)
        return 0

    lax.fori_loop(0, ROWS_PER_WORKER // NBUF - 1, body, 0)
    for b in range(NBUF):
        r = ROWS_PER_WORKER - NBUF + b
        wait(r, b)
        accum(r, b)
    pltpu.sync_copy(out_stage, out_hbm.at[pl.ds(base, ROWS_PER_WORKER)])


def _embedding_bag(text3, emb_bf):
    mesh = plsc.VectorSubcoreMesh(core_axis_name="c", subcore_axis_name="s")
    run = pl.kernel(
        _bag_body,
        out_type=jax.ShapeDtypeStruct((BATCH, EMB_DIM // 2), jnp.int32),
        mesh=mesh,
        compiler_params=pltpu.CompilerParams(use_tc_tiling_on_sc=False),
        scratch_types=[
            pltpu.VMEM((ROWS_PER_WORKER, 2, HALF_SEQ), jnp.int32),
            pltpu.VMEM((NBUF, 2, HALF_SEQ, EMB_DIM // 2), jnp.int32),
            pltpu.VMEM((ROWS_PER_WORKER, EMB_DIM // 2), jnp.int32),
            pltpu.SemaphoreType.DMA,
            pltpu.SemaphoreType.DMA,
            pltpu.SemaphoreType.DMA,
            pltpu.SemaphoreType.DMA,
        ],
    )
    return run(text3, emb_bf)


def _mlp_body(x_ref, w1_ref, b1_ref, w2_ref, b2_ref, wc_ref, bc_ref, out_ref):
    x = jnp.tanh(x_ref[...].astype(jnp.float32))
    h1 = jnp.tanh(jnp.dot(x, w1_ref[...].T, preferred_element_type=jnp.float32) + b1_ref[...])
    h2 = jnp.tanh(jnp.dot(h1, w2_ref[...].T, preferred_element_type=jnp.float32) + b2_ref[...])
    logits = jnp.dot(h2, wc_ref[...].T, preferred_element_type=jnp.float32) + bc_ref[...]
    m = jnp.max(logits, axis=-1, keepdims=True)
    e = jnp.exp(logits - m)
    out_ref[...] = e / jnp.sum(e, axis=-1, keepdims=True)


def _mlp(summed, W1, b1, W2, b2, Wc, bc):
    blk = 512
    grid = (BATCH // blk,)
    full = lambda shape: pl.BlockSpec(shape, lambda i: (0,) * len(shape))
    return pl.pallas_call(
        _mlp_body,
        grid=grid,
        in_specs=[
            pl.BlockSpec((blk, EMB_DIM), lambda i: (i, 0)),
            full(W1.shape),
            full(b1.shape),
            full(W2.shape),
            full(b2.shape),
            full(Wc.shape),
            full(bc.shape),
        ],
        out_specs=pl.BlockSpec((blk, 2), lambda i: (i, 0)),
        out_shape=jax.ShapeDtypeStruct((BATCH, 2), jnp.float32),
    )(summed, W1, b1, W2, b2, Wc, bc)


def kernel(text, emb, W1, b1, W2, b2, Wc, bc):
    text3 = text.astype(jnp.int32).reshape(BATCH, 2, HALF_SEQ)
    emb_bf = emb.astype(jnp.bfloat16)
    emb_words = lax.bitcast_convert_type(
        emb_bf.reshape(emb.shape[0], EMB_DIM // 2, 2), jnp.int32)
    summed_words = _embedding_bag(text3, emb_words)
    summed = lax.bitcast_convert_type(
        summed_words, jnp.bfloat16).reshape(BATCH, EMB_DIM)
    b1r = b1.reshape(1, -1)
    b2r = b2.reshape(1, -1)
    bcr = bc.reshape(1, -1)
    return _mlp(summed, W1, b1r, W2, b2r, Wc, bcr)
